# baseline (speedup 1.0000x reference)
import jax
import jax.numpy as jnp
from jax.experimental import pallas as pl


def _vq_block_kernel(z_ref, emb_ref, embn2_ref, out_ref):
    zc, zh, zw = z_ref.shape[1:]
    z = z_ref[0].reshape(zc, zh * zw)
    emb = emb_ref[...]
    embn2 = embn2_ref[...]
    k = emb.shape[0]

    e_sq = jnp.sum(emb * emb, axis=1, keepdims=True)
    scores = jax.lax.dot_general(
        embn2, z, (((1,), (0,)), ((), ())),
        preferred_element_type=jnp.float32)
    d2 = scores + e_sq

    minv = jnp.min(d2, axis=0, keepdims=True)
    rows = jax.lax.broadcasted_iota(jnp.int32, d2.shape, 0)
    idx = jnp.min(jnp.where(d2 == minv, rows, k), axis=0, keepdims=True)

    onehot = (rows == idx).astype(jnp.float32)
    zq = jax.lax.dot_general(
        emb, onehot, (((0,), (0,)), ((), ())),
        preferred_element_type=jnp.float32)
    out_ref[0] = zq.reshape(zc, zh, zw)


def kernel(z_e, embedding):
    B, C, H, W = z_e.shape
    K = embedding.shape[0]
    out = pl.pallas_call(
        _vq_block_kernel,
        grid=(B,),
        in_specs=[
            pl.BlockSpec((1, C, H, W), lambda b: (b, 0, 0, 0)),
            pl.BlockSpec((K, C), lambda b: (0, 0)),
            pl.BlockSpec((K, C), lambda b: (0, 0)),
        ],
        out_specs=pl.BlockSpec((1, C, H, W), lambda b: (b, 0, 0, 0)),
        out_shape=jax.ShapeDtypeStruct((B, C, H, W), jnp.float32),
    )(z_e, embedding, -2.0 * embedding)
    return out


# speedup vs baseline: 1.8994x; 1.8994x over previous
import jax
import jax.numpy as jnp
from jax.experimental import pallas as pl


def _vq_block_kernel(z_ref, emb_ref, embn2_ref, out_ref):
    z = z_ref[0]
    emb = emb_ref[...]
    embn2 = embn2_ref[...]
    k = emb.shape[0]

    e_sq = jnp.sum(emb * emb, axis=1, keepdims=True)
    scores = jax.lax.dot_general(
        embn2, z, (((1,), (0,)), ((), ())),
        preferred_element_type=jnp.float32)
    d2 = scores + e_sq

    rows = jax.lax.broadcasted_iota(jnp.int32, d2.shape, 0)
    idx = jnp.argmin(d2, axis=0)[None, :]

    onehot = (rows == idx).astype(jnp.float32)
    out_ref[0] = jax.lax.dot_general(
        emb, onehot, (((0,), (0,)), ((), ())),
        preferred_element_type=jnp.float32)


def kernel(z_e, embedding):
    B, C, H, W = z_e.shape
    K = embedding.shape[0]
    T = H * W
    z = z_e.reshape(B, C, T)
    out = pl.pallas_call(
        _vq_block_kernel,
        grid=(B,),
        in_specs=[
            pl.BlockSpec((1, C, T), lambda b: (b, 0, 0)),
            pl.BlockSpec((K, C), lambda b: (0, 0)),
            pl.BlockSpec((K, C), lambda b: (0, 0)),
        ],
        out_specs=pl.BlockSpec((1, C, T), lambda b: (b, 0, 0)),
        out_shape=jax.ShapeDtypeStruct((B, C, T), jnp.float32),
    )(z, embedding, -2.0 * embedding)
    return out.reshape(B, C, H, W)
